# Initial kernel scaffold; baseline (speedup 1.0000x reference)
import jax
import jax.numpy as jnp
from jax.experimental import pallas as pl


def kernel(x, adj_t, W1l, b1l, W1r, W2l, b2l, W2r):
    raise NotImplementedError("write your pallas kernel here")


# speedup vs baseline: 5.0646x; 5.0646x over previous
import functools

import jax
import jax.numpy as jnp
from jax import lax
from jax.experimental import pallas as pl
from jax.experimental.pallas import tpu as pltpu
from jax.experimental.pallas import tpu_sc as plsc

N = 10000
E = 320000
D = 128
NP = 10240

NC = 2
NS = 16
NW = NC * NS
E_PER_CORE = E // NC
E_PER_W = E // NW
CHUNK = 80
NCHUNK = E_PER_W // CHUNK
ROWS_PER_TILE = NP // NS

_mesh = plsc.VectorSubcoreMesh(core_axis_name="c", subcore_axis_name="s")


def _make_agg(with_deg):
    out_type = [jax.ShapeDtypeStruct((NC, NP, D), jnp.float32)]
    scratch = [
        pltpu.VMEM_SHARED((NP, D), jnp.float32),
        pltpu.VMEM((CHUNK,), jnp.int32),
        pltpu.VMEM((CHUNK,), jnp.int32),
        pltpu.VMEM((CHUNK, D), jnp.float32),
    ]
    if with_deg:
        out_type += [jax.ShapeDtypeStruct((NP,), jnp.float32),
                     jax.ShapeDtypeStruct((NP,), jnp.float32)]
        scratch += [
            pltpu.VMEM_SHARED((NP,), jnp.float32),
            pltpu.VMEM((CHUNK,), jnp.float32),
        ]

    def body(*refs):
        if with_deg:
            (table, src, dst, zrows, zdeg, out_agg, out_deg0, out_deg1,
             acc, src_v, dst_v, rows_v, dega, ones_v) = refs
        else:
            (table, src, dst, zrows, out_agg,
             acc, src_v, dst_v, rows_v) = refs
        c = lax.axis_index("c")
        s = lax.axis_index("s")
        r0 = s * ROWS_PER_TILE

        pltpu.sync_copy(zrows.at[pl.ds(r0, ROWS_PER_TILE)],
                        acc.at[pl.ds(r0, ROWS_PER_TILE)])
        if with_deg:
            @pl.when(s == 0)
            def _():
                pltpu.sync_copy(zdeg, dega)
            for j in range(CHUNK // 16):
                ones_v[pl.ds(j * 16, 16)] = jnp.full((16,), 1.0, jnp.float32)
        plsc.subcore_barrier()

        base_w = c * E_PER_CORE + s * E_PER_W

        def step(i, carry):
            base = base_w + i * CHUNK
            pltpu.sync_copy(src.at[pl.ds(base, CHUNK)], src_v)
            pltpu.sync_copy(dst.at[pl.ds(base, CHUNK)], dst_v)
            pltpu.sync_copy(table.at[src_v], rows_v)
            pltpu.sync_copy(rows_v, acc.at[dst_v], add=True)
            if with_deg:
                pltpu.sync_copy(ones_v, dega.at[dst_v], add=True)
            return carry

        lax.fori_loop(0, NCHUNK, step, 0)
        plsc.subcore_barrier()

        pltpu.sync_copy(acc.at[pl.ds(r0, ROWS_PER_TILE)],
                        out_agg.at[c, pl.ds(r0, ROWS_PER_TILE)])
        if with_deg:
            @pl.when(c == 0)
            def _():
                pltpu.sync_copy(dega.at[pl.ds(r0, ROWS_PER_TILE)],
                                out_deg0.at[pl.ds(r0, ROWS_PER_TILE)])

            @pl.when(c == 1)
            def _():
                pltpu.sync_copy(dega.at[pl.ds(r0, ROWS_PER_TILE)],
                                out_deg1.at[pl.ds(r0, ROWS_PER_TILE)])

    return pl.kernel(body, mesh=_mesh, out_type=tuple(out_type),
                     scratch_types=scratch)


_agg_deg = _make_agg(with_deg=True)
_agg = _make_agg(with_deg=False)

_R = 1024
_RS = _R // D


def _dense_body(p_ref, d0_ref, d1_ref, h_ref, wl_ref, bl_ref, wr_ref, o_ref,
                *, selu):
    agg = p_ref[0] + p_ref[1]
    deg = d0_ref[...] + d1_ref[...]
    r = 1.0 / jnp.maximum(deg, 1.0)
    a3 = agg.reshape(_RS, D, D) * r[:, :, None]
    a = a3.reshape(_R, D)
    out = (lax.dot_general(a, wl_ref[...], (((1,), (1,)), ((), ())),
                           preferred_element_type=jnp.float32)
           + bl_ref[...]
           + lax.dot_general(h_ref[...], wr_ref[...], (((1,), (1,)), ((), ())),
                             preferred_element_type=jnp.float32))
    if selu:
        alpha = 1.6732632423543772
        scale = 1.0507009873554805
        out = scale * jnp.where(out > 0, out, alpha * (jnp.exp(out) - 1.0))
    o_ref[...] = out


def _dense(p, d0, d1, h, Wl, bl2, Wr, selu):
    return pl.pallas_call(
        functools.partial(_dense_body, selu=selu),
        grid=(NP // _R,),
        in_specs=[
            pl.BlockSpec((NC, _R, D), lambda i: (0, i, 0)),
            pl.BlockSpec((_RS, D), lambda i: (i, 0)),
            pl.BlockSpec((_RS, D), lambda i: (i, 0)),
            pl.BlockSpec((_R, D), lambda i: (i, 0)),
            pl.BlockSpec((D, D), lambda i: (0, 0)),
            pl.BlockSpec((1, D), lambda i: (0, 0)),
            pl.BlockSpec((D, D), lambda i: (0, 0)),
        ],
        out_specs=pl.BlockSpec((_R, D), lambda i: (i, 0)),
        out_shape=jax.ShapeDtypeStruct((NP, D), jnp.float32),
    )(p, d0, d1, h, Wl, bl2, Wr)


def kernel(x, adj_t, W1l, b1l, W1r, W2l, b2l, W2r):
    src = adj_t[0]
    dst = adj_t[1]
    xp = jnp.pad(x, ((0, NP - N), (0, 0)))
    zrows = jnp.zeros((NP, D), jnp.float32)
    zdeg = jnp.zeros((NP,), jnp.float32)
    p1, deg0, deg1 = _agg_deg(xp, src, dst, zrows, zdeg)
    d0 = deg0.reshape(NP // D, D)
    d1 = deg1.reshape(NP // D, D)
    h1 = _dense(p1, d0, d1, xp, W1l, b1l.reshape(1, D), W1r, selu=True)
    p2, = _agg(h1, src, dst, zrows)
    out = _dense(p2, d0, d1, h1, W2l, b2l.reshape(1, D), W2r, selu=False)
    return out[:N]
